# Initial kernel scaffold; baseline (speedup 1.0000x reference)
import jax
import jax.numpy as jnp
from jax.experimental import pallas as pl


def kernel(x, edge_index, W1l, b1, W1r, g1, be1, W2l, b2, W2r, g2, be2):
    raise NotImplementedError("write your pallas kernel here")


# speedup vs baseline: 4.7567x; 4.7567x over previous
import functools

import jax
import jax.numpy as jnp
from jax import lax
from jax.experimental import pallas as pl
from jax.experimental.pallas import tpu as pltpu
from jax.experimental.pallas import tpu_sc as plsc

N = 10000
E = 320000
D = 128
EPS = 1e-5

NC = 2
NS = 16
NW = NC * NS
EPW = E // NW
B = 80
NITER = EPW // B
NP_ = 10240
ROWS = NP_ // NS
NCHUNK = ROWS // B

_mesh = plsc.VectorSubcoreMesh(core_axis_name="c", subcore_axis_name="s",
                               num_cores=NC, num_subcores=NS)


def _fill(ref, nrows, width, value):
    v = jnp.full((16,), value, jnp.float32)

    def row(i, carry):
        for j in range(width // 16):
            ref[i, pl.ds(j * 16, 16)] = v
        return carry

    lax.fori_loop(0, nrows, row, 0)


def _zero_acc(rows_v, acc_s, r0):
    _fill(rows_v, B, D, 0.0)
    for k in range(NCHUNK):
        pltpu.sync_copy(rows_v, acc_s.at[pl.ds(r0 + k * B, B)])


def _writeback(rows_v, acc_s, out_hbm, r0, wb):
    for k in range(NCHUNK):
        pltpu.sync_copy(acc_s.at[pl.ds(r0 + k * B, B)], rows_v)
        pltpu.sync_copy(rows_v, out_hbm.at[pl.ds(wb + k * B, B)])


@functools.partial(
    pl.kernel,
    out_type=(
        jax.ShapeDtypeStruct((NC * NP_, D), jnp.float32),
        jax.ShapeDtypeStruct((NC * NP_, D), jnp.float32),
    ),
    mesh=_mesh,
    scratch_types=[
        pltpu.VMEM((B,), jnp.int32),
        pltpu.VMEM((B,), jnp.int32),
        pltpu.VMEM((B, D), jnp.float32),
        pltpu.VMEM_SHARED((NP_, D), jnp.float32),
        pltpu.SemaphoreType.DMA,
    ],
)
def _segsum_cnt(y_hbm, src_hbm, dst_hbm,
                acc_out, cnt_out,
                src_v, dst_v, rows_v, acc_s, sem):
    c = lax.axis_index("c")
    s = lax.axis_index("s")
    wid = c * NS + s
    r0 = s * ROWS
    wb = c * NP_ + r0
    base = wid * EPW

    _zero_acc(rows_v, acc_s, r0)
    plsc.subcore_barrier()

    def body(i, carry):
        off = base + i * B
        pltpu.sync_copy(src_hbm.at[pl.ds(off, B)], src_v)
        pltpu.sync_copy(dst_hbm.at[pl.ds(off, B)], dst_v)
        pltpu.async_copy(y_hbm.at[src_v], rows_v, sem).wait()
        pltpu.sync_copy(rows_v, acc_s.at[dst_v], add=True)
        return carry

    lax.fori_loop(0, NITER, body, 0)
    plsc.subcore_barrier()
    _writeback(rows_v, acc_s, acc_out, r0, wb)
    plsc.subcore_barrier()

    _zero_acc(rows_v, acc_s, r0)
    plsc.subcore_barrier()
    _fill(rows_v, B, D, 1.0)

    def body_cnt(i, carry):
        off = base + i * B
        pltpu.sync_copy(dst_hbm.at[pl.ds(off, B)], dst_v)
        pltpu.sync_copy(rows_v, acc_s.at[dst_v], add=True)
        return carry

    lax.fori_loop(0, NITER, body_cnt, 0)
    plsc.subcore_barrier()
    _writeback(rows_v, acc_s, cnt_out, r0, wb)


@functools.partial(
    pl.kernel,
    out_type=jax.ShapeDtypeStruct((NC * NP_, D), jnp.float32),
    mesh=_mesh,
    scratch_types=[
        pltpu.VMEM((B,), jnp.int32),
        pltpu.VMEM((B,), jnp.int32),
        pltpu.VMEM((B, D), jnp.float32),
        pltpu.VMEM_SHARED((NP_, D), jnp.float32),
        pltpu.SemaphoreType.DMA,
    ],
)
def _segsum(y_hbm, src_hbm, dst_hbm,
            acc_out,
            src_v, dst_v, rows_v, acc_s, sem):
    c = lax.axis_index("c")
    s = lax.axis_index("s")
    wid = c * NS + s
    r0 = s * ROWS
    wb = c * NP_ + r0
    base = wid * EPW

    _zero_acc(rows_v, acc_s, r0)
    plsc.subcore_barrier()

    def body(i, carry):
        off = base + i * B
        pltpu.sync_copy(src_hbm.at[pl.ds(off, B)], src_v)
        pltpu.sync_copy(dst_hbm.at[pl.ds(off, B)], dst_v)
        pltpu.async_copy(y_hbm.at[src_v], rows_v, sem).wait()
        pltpu.sync_copy(rows_v, acc_s.at[dst_v], add=True)
        return carry

    lax.fori_loop(0, NITER, body, 0)
    plsc.subcore_barrier()
    _writeback(rows_v, acc_s, acc_out, r0, wb)


RB = 1000
NBLK = N // RB


def _dense_body(p0, p1, c0, c1, y, wl, wr, b, z, stats):
    i = pl.program_id(0)
    cnt = jnp.maximum(c0[0, :, 0:1] + c1[0, :, 0:1], 1.0)
    mean = (p0[0] + p1[0]) / cnt
    zz = (jnp.dot(mean, wl[...], preferred_element_type=jnp.float32)
          + jnp.dot(y[...], wr[...], preferred_element_type=jnp.float32)
          + b[...])
    z[...] = zz

    @pl.when(i == 0)
    def _():
        stats[...] = jnp.zeros_like(stats)

    stats[0:1, :] += jnp.sum(zz, axis=0, keepdims=True)
    stats[1:2, :] += jnp.sum(zz * zz, axis=0, keepdims=True)


_dense = pl.pallas_call(
    _dense_body,
    grid=(NBLK,),
    in_specs=[
        pl.BlockSpec((1, RB, D), lambda i: (0, i, 0)),
        pl.BlockSpec((1, RB, D), lambda i: (1, i, 0)),
        pl.BlockSpec((1, RB, D), lambda i: (0, i, 0)),
        pl.BlockSpec((1, RB, D), lambda i: (1, i, 0)),
        pl.BlockSpec((RB, D), lambda i: (i, 0)),
        pl.BlockSpec((D, D), lambda i: (0, 0)),
        pl.BlockSpec((D, D), lambda i: (0, 0)),
        pl.BlockSpec((1, D), lambda i: (0, 0)),
    ],
    out_specs=[
        pl.BlockSpec((RB, D), lambda i: (i, 0)),
        pl.BlockSpec((8, D), lambda i: (0, 0)),
    ],
    out_shape=[
        jax.ShapeDtypeStruct((N, D), jnp.float32),
        jax.ShapeDtypeStruct((8, D), jnp.float32),
    ],
)


def _affine_body(z, stats, g, be, res, out):
    mu = stats[0:1, :] * (1.0 / N)
    var = stats[1:2, :] * (1.0 / N) - mu * mu
    a = g[...] * lax.rsqrt(var + EPS)
    c = be[...] - mu * a
    o = z[...] * a + c
    if res is not None:
        o = o + res[...]
    out[...] = o


_affine = pl.pallas_call(
    lambda z, stats, g, be, out: _affine_body(z, stats, g, be, None, out),
    grid=(NBLK,),
    in_specs=[
        pl.BlockSpec((RB, D), lambda i: (i, 0)),
        pl.BlockSpec((8, D), lambda i: (0, 0)),
        pl.BlockSpec((1, D), lambda i: (0, 0)),
        pl.BlockSpec((1, D), lambda i: (0, 0)),
    ],
    out_specs=pl.BlockSpec((RB, D), lambda i: (i, 0)),
    out_shape=jax.ShapeDtypeStruct((N, D), jnp.float32),
)

_affine_res = pl.pallas_call(
    _affine_body,
    grid=(NBLK,),
    in_specs=[
        pl.BlockSpec((RB, D), lambda i: (i, 0)),
        pl.BlockSpec((8, D), lambda i: (0, 0)),
        pl.BlockSpec((1, D), lambda i: (0, 0)),
        pl.BlockSpec((1, D), lambda i: (0, 0)),
        pl.BlockSpec((RB, D), lambda i: (i, 0)),
    ],
    out_specs=pl.BlockSpec((RB, D), lambda i: (i, 0)),
    out_shape=jax.ShapeDtypeStruct((N, D), jnp.float32),
)


def kernel(x, edge_index, W1l, b1, W1r, g1, be1, W2l, b2, W2r, g2, be2):
    src = edge_index[0]
    dst = edge_index[1]

    acc1, cnt = _segsum_cnt(x, src, dst)
    acc1 = acc1.reshape(NC, NP_, D)
    cnt = cnt.reshape(NC, NP_, D)
    z1, s1 = _dense(acc1, acc1, cnt, cnt, x,
                    W1l.T, W1r.T, b1.reshape(1, D))
    h1 = _affine(z1, s1, g1.reshape(1, D), be1.reshape(1, D))

    acc2 = _segsum(h1, src, dst).reshape(NC, NP_, D)
    z2, s2 = _dense(acc2, acc2, cnt, cnt, h1,
                    W2l.T, W2r.T, b2.reshape(1, D))
    return _affine_res(z2, s2, g2.reshape(1, D), be2.reshape(1, D), x)


# speedup vs baseline: 7.6054x; 1.5989x over previous
import functools

import jax
import jax.numpy as jnp
from jax import lax
from jax.experimental import pallas as pl
from jax.experimental.pallas import tpu as pltpu
from jax.experimental.pallas import tpu_sc as plsc

N = 10000
E = 320000
D = 128
EPS = 1e-5

NC = 2
NS = 16
NW = NC * NS
EPW = E // NW
B = 80
NITER = EPW // B
NP_ = 10240
ROWS = NP_ // NS
NCHUNK = ROWS // B

_mesh = plsc.VectorSubcoreMesh(core_axis_name="c", subcore_axis_name="s",
                               num_cores=NC, num_subcores=NS)


def _fill(ref, nrows, width, value):
    v = jnp.full((16,), value, jnp.float32)

    def row(i, carry):
        for j in range(width // 16):
            ref[i, pl.ds(j * 16, 16)] = v
        return carry

    lax.fori_loop(0, nrows, row, 0)


def _zero_acc(rows_v, acc_s, r0):
    _fill(rows_v, B, D, 0.0)
    for k in range(NCHUNK):
        pltpu.sync_copy(rows_v, acc_s.at[pl.ds(r0 + k * B, B)])


def _writeback(rows_v, acc_s, out_hbm, r0, wb):
    for k in range(NCHUNK):
        pltpu.sync_copy(acc_s.at[pl.ds(r0 + k * B, B)], rows_v)
        pltpu.sync_copy(rows_v, out_hbm.at[pl.ds(wb + k * B, B)])


def _gather_segsum(y_hbm, src_hbm, dst_hbm, acc_s, base,
                   src_v0, dst_v0, rows_v0, sem0,
                   src_v1, dst_v1, rows_v1, sem1):
    def load_idx(off, src_v, dst_v):
        pltpu.sync_copy(src_hbm.at[pl.ds(off, B)], src_v)
        pltpu.sync_copy(dst_hbm.at[pl.ds(off, B)], dst_v)

    def drain(src_v, rows_v, sem):
        pltpu.make_async_copy(y_hbm.at[src_v], rows_v, sem).wait()

    load_idx(base, src_v0, dst_v0)
    pltpu.async_copy(y_hbm.at[src_v0], rows_v0, sem0)

    def body(g, carry):
        load_idx(base + (2 * g + 1) * B, src_v1, dst_v1)
        pltpu.async_copy(y_hbm.at[src_v1], rows_v1, sem1)
        drain(src_v0, rows_v0, sem0)
        pltpu.sync_copy(rows_v0, acc_s.at[dst_v0], add=True)
        load_idx(base + (2 * g + 2) * B, src_v0, dst_v0)
        pltpu.async_copy(y_hbm.at[src_v0], rows_v0, sem0)
        drain(src_v1, rows_v1, sem1)
        pltpu.sync_copy(rows_v1, acc_s.at[dst_v1], add=True)
        return carry

    lax.fori_loop(0, (NITER - 1) // 2, body, 0)
    drain(src_v0, rows_v0, sem0)
    pltpu.sync_copy(rows_v0, acc_s.at[dst_v0], add=True)


def _count_pass(dst_hbm, acc_s, base, ones_v,
                dst_v0, sem0, dst_v1, sem1):
    def drain(dst_v, sem):
        pltpu.make_async_copy(dst_hbm.at[pl.ds(base, B)], dst_v, sem).wait()

    pltpu.async_copy(dst_hbm.at[pl.ds(base, B)], dst_v0, sem0)

    def body(g, carry):
        pltpu.async_copy(dst_hbm.at[pl.ds(base + (2 * g + 1) * B, B)],
                         dst_v1, sem1)
        drain(dst_v0, sem0)
        pltpu.sync_copy(ones_v, acc_s.at[dst_v0], add=True)
        pltpu.async_copy(dst_hbm.at[pl.ds(base + (2 * g + 2) * B, B)],
                         dst_v0, sem0)
        drain(dst_v1, sem1)
        pltpu.sync_copy(ones_v, acc_s.at[dst_v1], add=True)
        return carry

    lax.fori_loop(0, (NITER - 1) // 2, body, 0)
    drain(dst_v0, sem0)
    pltpu.sync_copy(ones_v, acc_s.at[dst_v0], add=True)


_SC_SCRATCH = [
    pltpu.VMEM((B,), jnp.int32),
    pltpu.VMEM((B,), jnp.int32),
    pltpu.VMEM((B, D), jnp.float32),
    pltpu.VMEM((B,), jnp.int32),
    pltpu.VMEM((B,), jnp.int32),
    pltpu.VMEM((B, D), jnp.float32),
    pltpu.VMEM_SHARED((NP_, D), jnp.float32),
    pltpu.SemaphoreType.DMA,
    pltpu.SemaphoreType.DMA,
]


@functools.partial(
    pl.kernel,
    out_type=(
        jax.ShapeDtypeStruct((NC * NP_, D), jnp.float32),
        jax.ShapeDtypeStruct((NC * NP_, D), jnp.float32),
    ),
    mesh=_mesh,
    scratch_types=list(_SC_SCRATCH),
)
def _segsum_cnt(y_hbm, src_hbm, dst_hbm,
                acc_out, cnt_out,
                src_v0, dst_v0, rows_v0, src_v1, dst_v1, rows_v1,
                acc_s, sem0, sem1):
    c = lax.axis_index("c")
    s = lax.axis_index("s")
    wid = c * NS + s
    r0 = s * ROWS
    wb = c * NP_ + r0
    base = wid * EPW

    _zero_acc(rows_v0, acc_s, r0)
    plsc.subcore_barrier()
    _gather_segsum(y_hbm, src_hbm, dst_hbm, acc_s, base,
                   src_v0, dst_v0, rows_v0, sem0,
                   src_v1, dst_v1, rows_v1, sem1)
    plsc.subcore_barrier()
    _writeback(rows_v0, acc_s, acc_out, r0, wb)
    plsc.subcore_barrier()

    _zero_acc(rows_v0, acc_s, r0)
    plsc.subcore_barrier()
    _fill(rows_v0, B, D, 1.0)
    _count_pass(dst_hbm, acc_s, base, rows_v0, dst_v0, sem0, dst_v1, sem1)
    plsc.subcore_barrier()
    _writeback(rows_v1, acc_s, cnt_out, r0, wb)


@functools.partial(
    pl.kernel,
    out_type=jax.ShapeDtypeStruct((NC * NP_, D), jnp.float32),
    mesh=_mesh,
    scratch_types=list(_SC_SCRATCH),
)
def _segsum(y_hbm, src_hbm, dst_hbm,
            acc_out,
            src_v0, dst_v0, rows_v0, src_v1, dst_v1, rows_v1,
            acc_s, sem0, sem1):
    c = lax.axis_index("c")
    s = lax.axis_index("s")
    wid = c * NS + s
    r0 = s * ROWS
    wb = c * NP_ + r0
    base = wid * EPW

    _zero_acc(rows_v0, acc_s, r0)
    plsc.subcore_barrier()
    _gather_segsum(y_hbm, src_hbm, dst_hbm, acc_s, base,
                   src_v0, dst_v0, rows_v0, sem0,
                   src_v1, dst_v1, rows_v1, sem1)
    plsc.subcore_barrier()
    _writeback(rows_v0, acc_s, acc_out, r0, wb)


RB = 1000
NBLK = N // RB


def _dense_body(p0, p1, c0, c1, y, wl, wr, b, z, stats):
    i = pl.program_id(0)
    cnt = jnp.maximum(c0[0, :, 0:1] + c1[0, :, 0:1], 1.0)
    mean = (p0[0] + p1[0]) / cnt
    zz = (jnp.dot(mean, wl[...], preferred_element_type=jnp.float32)
          + jnp.dot(y[...], wr[...], preferred_element_type=jnp.float32)
          + b[...])
    z[...] = zz

    @pl.when(i == 0)
    def _():
        stats[...] = jnp.zeros_like(stats)

    stats[0:1, :] += jnp.sum(zz, axis=0, keepdims=True)
    stats[1:2, :] += jnp.sum(zz * zz, axis=0, keepdims=True)


_dense = pl.pallas_call(
    _dense_body,
    grid=(NBLK,),
    in_specs=[
        pl.BlockSpec((1, RB, D), lambda i: (0, i, 0)),
        pl.BlockSpec((1, RB, D), lambda i: (1, i, 0)),
        pl.BlockSpec((1, RB, D), lambda i: (0, i, 0)),
        pl.BlockSpec((1, RB, D), lambda i: (1, i, 0)),
        pl.BlockSpec((RB, D), lambda i: (i, 0)),
        pl.BlockSpec((D, D), lambda i: (0, 0)),
        pl.BlockSpec((D, D), lambda i: (0, 0)),
        pl.BlockSpec((1, D), lambda i: (0, 0)),
    ],
    out_specs=[
        pl.BlockSpec((RB, D), lambda i: (i, 0)),
        pl.BlockSpec((8, D), lambda i: (0, 0)),
    ],
    out_shape=[
        jax.ShapeDtypeStruct((N, D), jnp.float32),
        jax.ShapeDtypeStruct((8, D), jnp.float32),
    ],
)


def _affine_body(z, stats, g, be, res, out):
    mu = stats[0:1, :] * (1.0 / N)
    var = stats[1:2, :] * (1.0 / N) - mu * mu
    a = g[...] * lax.rsqrt(var + EPS)
    c = be[...] - mu * a
    o = z[...] * a + c
    if res is not None:
        o = o + res[...]
    out[...] = o


_affine = pl.pallas_call(
    lambda z, stats, g, be, out: _affine_body(z, stats, g, be, None, out),
    grid=(NBLK,),
    in_specs=[
        pl.BlockSpec((RB, D), lambda i: (i, 0)),
        pl.BlockSpec((8, D), lambda i: (0, 0)),
        pl.BlockSpec((1, D), lambda i: (0, 0)),
        pl.BlockSpec((1, D), lambda i: (0, 0)),
    ],
    out_specs=pl.BlockSpec((RB, D), lambda i: (i, 0)),
    out_shape=jax.ShapeDtypeStruct((N, D), jnp.float32),
)

_affine_res = pl.pallas_call(
    _affine_body,
    grid=(NBLK,),
    in_specs=[
        pl.BlockSpec((RB, D), lambda i: (i, 0)),
        pl.BlockSpec((8, D), lambda i: (0, 0)),
        pl.BlockSpec((1, D), lambda i: (0, 0)),
        pl.BlockSpec((1, D), lambda i: (0, 0)),
        pl.BlockSpec((RB, D), lambda i: (i, 0)),
    ],
    out_specs=pl.BlockSpec((RB, D), lambda i: (i, 0)),
    out_shape=jax.ShapeDtypeStruct((N, D), jnp.float32),
)


def kernel(x, edge_index, W1l, b1, W1r, g1, be1, W2l, b2, W2r, g2, be2):
    src = edge_index[0]
    dst = edge_index[1]

    acc1, cnt = _segsum_cnt(x, src, dst)
    acc1 = acc1.reshape(NC, NP_, D)
    cnt = cnt.reshape(NC, NP_, D)
    z1, s1 = _dense(acc1, acc1, cnt, cnt, x,
                    W1l.T, W1r.T, b1.reshape(1, D))
    h1 = _affine(z1, s1, g1.reshape(1, D), be1.reshape(1, D))

    acc2 = _segsum(h1, src, dst).reshape(NC, NP_, D)
    z2, s2 = _dense(acc2, acc2, cnt, cnt, h1,
                    W2l.T, W2r.T, b2.reshape(1, D))
    return _affine_res(z2, s2, g2.reshape(1, D), be2.reshape(1, D), x)


# speedup vs baseline: 7.6252x; 1.0026x over previous
import functools

import jax
import jax.numpy as jnp
from jax import lax
from jax.experimental import pallas as pl
from jax.experimental.pallas import tpu as pltpu
from jax.experimental.pallas import tpu_sc as plsc

N = 10000
E = 320000
D = 128
EPS = 1e-5

NC = 2
NS = 16
NW = NC * NS
EPW = E // NW
B = 80
NITER = EPW // B
NP_ = 10240
ROWS = NP_ // NS
NCHUNK = ROWS // B

_mesh = plsc.VectorSubcoreMesh(core_axis_name="c", subcore_axis_name="s",
                               num_cores=NC, num_subcores=NS)


def _fill(ref, nrows, width, value):
    v = jnp.full((16,), value, jnp.float32)

    def row(i, carry):
        for j in range(width // 16):
            ref[i, pl.ds(j * 16, 16)] = v
        return carry

    lax.fori_loop(0, nrows, row, 0)


def _zero_acc(rows_v, acc_s, r0):
    _fill(rows_v, B, D, 0.0)
    for k in range(NCHUNK):
        pltpu.sync_copy(rows_v, acc_s.at[pl.ds(r0 + k * B, B)])


def _writeback(rows_v, acc_s, out_hbm, r0, wb):
    for k in range(NCHUNK):
        pltpu.sync_copy(acc_s.at[pl.ds(r0 + k * B, B)], rows_v)
        pltpu.sync_copy(rows_v, out_hbm.at[pl.ds(wb + k * B, B)])


def _gather_segsum(y_hbm, src_hbm, dst_hbm, acc_s, base, bufs):
    def load_idx(off, src_v, dst_v):
        pltpu.sync_copy(src_hbm.at[pl.ds(off, B)], src_v)
        pltpu.sync_copy(dst_hbm.at[pl.ds(off, B)], dst_v)

    def issue(off, buf):
        src_v, dst_v, rows_v, sem = buf
        load_idx(off, src_v, dst_v)
        pltpu.async_copy(y_hbm.at[src_v], rows_v, sem)

    def drain_scatter(buf):
        src_v, dst_v, rows_v, sem = buf
        pltpu.make_async_copy(y_hbm.at[src_v], rows_v, sem).wait()
        pltpu.sync_copy(rows_v, acc_s.at[dst_v], add=True)

    issue(base, bufs[0])
    issue(base + B, bufs[1])

    def body(g, carry):
        for j in range(4):
            issue(base + (4 * g + j + 2) * B, bufs[(j + 2) % 4])
            drain_scatter(bufs[j])
        return carry

    lax.fori_loop(0, (NITER - 5) // 4, body, 0)
    issue(base + 122 * B, bufs[2])
    drain_scatter(bufs[0])
    issue(base + 123 * B, bufs[3])
    drain_scatter(bufs[1])
    issue(base + 124 * B, bufs[0])
    drain_scatter(bufs[2])
    drain_scatter(bufs[3])
    drain_scatter(bufs[0])


def _count_pass(dst_hbm, acc_s, base, ones_v,
                dst_v0, sem0, dst_v1, sem1):
    def drain(dst_v, sem):
        pltpu.make_async_copy(dst_hbm.at[pl.ds(base, B)], dst_v, sem).wait()

    pltpu.async_copy(dst_hbm.at[pl.ds(base, B)], dst_v0, sem0)

    def body(g, carry):
        pltpu.async_copy(dst_hbm.at[pl.ds(base + (2 * g + 1) * B, B)],
                         dst_v1, sem1)
        drain(dst_v0, sem0)
        pltpu.sync_copy(ones_v, acc_s.at[dst_v0], add=True)
        pltpu.async_copy(dst_hbm.at[pl.ds(base + (2 * g + 2) * B, B)],
                         dst_v0, sem0)
        drain(dst_v1, sem1)
        pltpu.sync_copy(ones_v, acc_s.at[dst_v1], add=True)
        return carry

    lax.fori_loop(0, (NITER - 1) // 2, body, 0)
    drain(dst_v0, sem0)
    pltpu.sync_copy(ones_v, acc_s.at[dst_v0], add=True)


_SC_SCRATCH = (
    [pltpu.VMEM((B,), jnp.int32),
     pltpu.VMEM((B,), jnp.int32),
     pltpu.VMEM((B, D), jnp.float32)] * 4
    + [pltpu.VMEM_SHARED((NP_, D), jnp.float32)]
    + [pltpu.SemaphoreType.DMA] * 4
)


@functools.partial(
    pl.kernel,
    out_type=(
        jax.ShapeDtypeStruct((NC * NP_, D), jnp.float32),
        jax.ShapeDtypeStruct((NC * NP_, D), jnp.float32),
    ),
    mesh=_mesh,
    scratch_types=list(_SC_SCRATCH),
)
def _segsum_cnt(y_hbm, src_hbm, dst_hbm,
                acc_out, cnt_out,
                src_v0, dst_v0, rows_v0, src_v1, dst_v1, rows_v1,
                src_v2, dst_v2, rows_v2, src_v3, dst_v3, rows_v3,
                acc_s, sem0, sem1, sem2, sem3):
    bufs = [(src_v0, dst_v0, rows_v0, sem0),
            (src_v1, dst_v1, rows_v1, sem1),
            (src_v2, dst_v2, rows_v2, sem2),
            (src_v3, dst_v3, rows_v3, sem3)]
    c = lax.axis_index("c")
    s = lax.axis_index("s")
    wid = c * NS + s
    r0 = s * ROWS
    wb = c * NP_ + r0
    base = wid * EPW

    _zero_acc(rows_v0, acc_s, r0)
    plsc.subcore_barrier()
    _gather_segsum(y_hbm, src_hbm, dst_hbm, acc_s, base, bufs)
    plsc.subcore_barrier()
    _writeback(rows_v0, acc_s, acc_out, r0, wb)
    plsc.subcore_barrier()

    _zero_acc(rows_v0, acc_s, r0)
    plsc.subcore_barrier()
    _fill(rows_v0, B, D, 1.0)
    _count_pass(dst_hbm, acc_s, base, rows_v0, dst_v0, sem0, dst_v1, sem1)
    plsc.subcore_barrier()
    _writeback(rows_v1, acc_s, cnt_out, r0, wb)


@functools.partial(
    pl.kernel,
    out_type=jax.ShapeDtypeStruct((NC * NP_, D), jnp.float32),
    mesh=_mesh,
    scratch_types=list(_SC_SCRATCH),
)
def _segsum(y_hbm, src_hbm, dst_hbm,
            acc_out,
            src_v0, dst_v0, rows_v0, src_v1, dst_v1, rows_v1,
            src_v2, dst_v2, rows_v2, src_v3, dst_v3, rows_v3,
            acc_s, sem0, sem1, sem2, sem3):
    bufs = [(src_v0, dst_v0, rows_v0, sem0),
            (src_v1, dst_v1, rows_v1, sem1),
            (src_v2, dst_v2, rows_v2, sem2),
            (src_v3, dst_v3, rows_v3, sem3)]
    c = lax.axis_index("c")
    s = lax.axis_index("s")
    wid = c * NS + s
    r0 = s * ROWS
    wb = c * NP_ + r0
    base = wid * EPW

    _zero_acc(rows_v0, acc_s, r0)
    plsc.subcore_barrier()
    _gather_segsum(y_hbm, src_hbm, dst_hbm, acc_s, base, bufs)
    plsc.subcore_barrier()
    _writeback(rows_v0, acc_s, acc_out, r0, wb)


RB = 1000
NBLK = N // RB


def _dense_body(p0, p1, c0, c1, y, wl, wr, b, z, stats):
    i = pl.program_id(0)
    cnt = jnp.maximum(c0[0, :, 0:1] + c1[0, :, 0:1], 1.0)
    mean = (p0[0] + p1[0]) / cnt
    zz = (jnp.dot(mean, wl[...], preferred_element_type=jnp.float32)
          + jnp.dot(y[...], wr[...], preferred_element_type=jnp.float32)
          + b[...])
    z[...] = zz

    @pl.when(i == 0)
    def _():
        stats[...] = jnp.zeros_like(stats)

    stats[0:1, :] += jnp.sum(zz, axis=0, keepdims=True)
    stats[1:2, :] += jnp.sum(zz * zz, axis=0, keepdims=True)


_dense = pl.pallas_call(
    _dense_body,
    grid=(NBLK,),
    in_specs=[
        pl.BlockSpec((1, RB, D), lambda i: (0, i, 0)),
        pl.BlockSpec((1, RB, D), lambda i: (1, i, 0)),
        pl.BlockSpec((1, RB, D), lambda i: (0, i, 0)),
        pl.BlockSpec((1, RB, D), lambda i: (1, i, 0)),
        pl.BlockSpec((RB, D), lambda i: (i, 0)),
        pl.BlockSpec((D, D), lambda i: (0, 0)),
        pl.BlockSpec((D, D), lambda i: (0, 0)),
        pl.BlockSpec((1, D), lambda i: (0, 0)),
    ],
    out_specs=[
        pl.BlockSpec((RB, D), lambda i: (i, 0)),
        pl.BlockSpec((8, D), lambda i: (0, 0)),
    ],
    out_shape=[
        jax.ShapeDtypeStruct((N, D), jnp.float32),
        jax.ShapeDtypeStruct((8, D), jnp.float32),
    ],
)


def _affine_body(z, stats, g, be, res, out):
    mu = stats[0:1, :] * (1.0 / N)
    var = stats[1:2, :] * (1.0 / N) - mu * mu
    a = g[...] * lax.rsqrt(var + EPS)
    c = be[...] - mu * a
    o = z[...] * a + c
    if res is not None:
        o = o + res[...]
    out[...] = o


_affine = pl.pallas_call(
    lambda z, stats, g, be, out: _affine_body(z, stats, g, be, None, out),
    grid=(NBLK,),
    in_specs=[
        pl.BlockSpec((RB, D), lambda i: (i, 0)),
        pl.BlockSpec((8, D), lambda i: (0, 0)),
        pl.BlockSpec((1, D), lambda i: (0, 0)),
        pl.BlockSpec((1, D), lambda i: (0, 0)),
    ],
    out_specs=pl.BlockSpec((RB, D), lambda i: (i, 0)),
    out_shape=jax.ShapeDtypeStruct((N, D), jnp.float32),
)

_affine_res = pl.pallas_call(
    _affine_body,
    grid=(NBLK,),
    in_specs=[
        pl.BlockSpec((RB, D), lambda i: (i, 0)),
        pl.BlockSpec((8, D), lambda i: (0, 0)),
        pl.BlockSpec((1, D), lambda i: (0, 0)),
        pl.BlockSpec((1, D), lambda i: (0, 0)),
        pl.BlockSpec((RB, D), lambda i: (i, 0)),
    ],
    out_specs=pl.BlockSpec((RB, D), lambda i: (i, 0)),
    out_shape=jax.ShapeDtypeStruct((N, D), jnp.float32),
)


def kernel(x, edge_index, W1l, b1, W1r, g1, be1, W2l, b2, W2r, g2, be2):
    src = edge_index[0]
    dst = edge_index[1]

    acc1, cnt = _segsum_cnt(x, src, dst)
    acc1 = acc1.reshape(NC, NP_, D)
    cnt = cnt.reshape(NC, NP_, D)
    z1, s1 = _dense(acc1, acc1, cnt, cnt, x,
                    W1l.T, W1r.T, b1.reshape(1, D))
    h1 = _affine(z1, s1, g1.reshape(1, D), be1.reshape(1, D))

    acc2 = _segsum(h1, src, dst).reshape(NC, NP_, D)
    z2, s2 = _dense(acc2, acc2, cnt, cnt, h1,
                    W2l.T, W2r.T, b2.reshape(1, D))
    return _affine_res(z2, s2, g2.reshape(1, D), be2.reshape(1, D), x)
